# Initial kernel scaffold; baseline (speedup 1.0000x reference)
import jax
import jax.numpy as jnp
from jax.experimental import pallas as pl


def kernel(positions, table):
    raise NotImplementedError("write your pallas kernel here")


# speedup vs baseline: 1.7526x; 1.7526x over previous
import functools

import jax
import jax.numpy as jnp
from jax import lax
from jax.experimental import pallas as pl
from jax.experimental.pallas import tpu as pltpu
from jax.experimental.pallas import tpu_sc as plsc

NC = 2
NS = 16
NW = NC * NS
CHUNK = 1024


def kernel(positions, table):
    B, H = positions.shape
    V, D = table.shape
    N = B * H
    per_w = N // NW
    n_chunks = per_w // CHUNK
    flat = positions.reshape(N)

    mesh = plsc.VectorSubcoreMesh(
        core_axis_name="c", subcore_axis_name="s", num_cores=NC, num_subcores=NS
    )

    @functools.partial(
        pl.kernel,
        out_type=jax.ShapeDtypeStruct((N, D), jnp.float32),
        mesh=mesh,
        scratch_types=[
            pltpu.VMEM((CHUNK,), jnp.int32),
            pltpu.VMEM((CHUNK, D), jnp.float32),
            pltpu.SemaphoreType.DMA,
        ],
        compiler_params=pltpu.CompilerParams(use_tc_tiling_on_sc=False),
    )
    def gather_kernel(pos_hbm, table_hbm, out_hbm, idx_v, rows_v, sem):
        wid = lax.axis_index("s") * NC + lax.axis_index("c")
        wbase = wid * per_w

        def body(g, carry):
            base = wbase + g * CHUNK
            pltpu.sync_copy(pos_hbm.at[pl.ds(base, CHUNK)], idx_v)
            pltpu.async_copy(table_hbm.at[idx_v], rows_v, sem).wait()
            pltpu.sync_copy(rows_v, out_hbm.at[pl.ds(base, CHUNK)])
            return carry

        lax.fori_loop(0, n_chunks, body, 0)

    out = gather_kernel(flat, table)
    return out.reshape(B, H, D)


# speedup vs baseline: 6.9559x; 3.9689x over previous
import functools

import jax
import jax.numpy as jnp
from jax import lax
from jax.experimental import pallas as pl
from jax.experimental.pallas import tpu as pltpu
from jax.experimental.pallas import tpu_sc as plsc

NC = 2
NS = 16
NW = NC * NS
NB = 4
CHUNK = 800


def kernel(positions, table):
    B, H = positions.shape
    V, D = table.shape
    N = B * H
    per_w = N // NW
    n_chunks = per_w // CHUNK
    n_outer = n_chunks // NB
    flat = positions.reshape(N)

    mesh = plsc.VectorSubcoreMesh(
        core_axis_name="c", subcore_axis_name="s", num_cores=NC, num_subcores=NS
    )

    @functools.partial(
        pl.kernel,
        out_type=jax.ShapeDtypeStruct((N, D), jnp.float32),
        mesh=mesh,
        scratch_types=[
            pltpu.VMEM_SHARED((V, D), jnp.float32),
            pltpu.VMEM((NB, CHUNK), jnp.int32),
            pltpu.VMEM((NB, CHUNK, D), jnp.float32),
            pltpu.SemaphoreType.DMA((NB,)),
            pltpu.SemaphoreType.DMA((NB,)),
        ],
        compiler_params=pltpu.CompilerParams(use_tc_tiling_on_sc=False),
    )
    def gather_kernel(pos_hbm, table_hbm, out_hbm, table_sh, idx_v, rows_v,
                      gsem, wsem):
        cid = lax.axis_index("c")
        sid = lax.axis_index("s")
        wid = sid * NC + cid
        wbase = wid * per_w

        @pl.when(sid == 0)
        def _():
            pltpu.sync_copy(table_hbm, table_sh)
        plsc.subcore_barrier()

        def start_gather(chunk, b):
            base = wbase + chunk * CHUNK
            pltpu.sync_copy(pos_hbm.at[pl.ds(base, CHUNK)], idx_v.at[b])
            pltpu.async_copy(table_sh.at[idx_v.at[b]], rows_v.at[b],
                             gsem.at[b])

        def wait_gather(b):
            pltpu.make_async_copy(out_hbm.at[pl.ds(0, CHUNK)], rows_v.at[b],
                                  gsem.at[b]).wait()

        def wait_writeback(chunk, b):
            base = wbase + chunk * CHUNK
            pltpu.make_async_copy(rows_v.at[b], out_hbm.at[pl.ds(base, CHUNK)],
                                  wsem.at[b]).wait()

        for b in range(NB - 1):
            start_gather(b, b)

        def body(outer, carry):
            for b in range(NB):
                g = outer * NB + b
                wait_gather(b)
                pltpu.async_copy(
                    rows_v.at[b], out_hbm.at[pl.ds(wbase + g * CHUNK, CHUNK)],
                    wsem.at[b])
                pre = g + NB - 1
                bp = (b + NB - 1) % NB

                @pl.when(pre < n_chunks)
                def _():
                    @pl.when(pre >= NB)
                    def _():
                        wait_writeback(pre - NB, bp)
                    start_gather(pre, bp)
            return carry

        lax.fori_loop(0, n_outer, body, 0)

        for b in range(NB):
            wait_writeback(n_chunks - NB + b, b)

    out = gather_kernel(flat, table)
    return out.reshape(B, H, D)
